# baseline (speedup 1.0000x reference)
import functools

import jax
import jax.numpy as jnp
from jax import lax
from jax.experimental import pallas as pl
from jax.experimental.pallas import tpu as pltpu
from jax.experimental.pallas import tpu_sc as plsc

N = 10000
E = 320000
D = 128
H = 32
G = 64
T = 10

EC = 1000
NWORKERS = 32
EPT = E // NWORKERS
NFC = EPT // EC
N_PAD = 10240
RPS = N_PAD // 16
ZR = 160
PR = N_PAD // 4


def _sc_agg_body(ei_hbm, h_hbm, out_hbm, acc, hbuf, sall, dall,
                 rows0, rows1, zbuf, gsem0, gsem1, ssem0, ssem1):
    rows = (rows0, rows1)
    gsem = (gsem0, gsem1)
    ssem = (ssem0, ssem1)
    cid = lax.axis_index("c")
    sid = lax.axis_index("s")
    wid = sid * 2 + cid

    jb = sid // 4
    r0 = (sid % 4) * RPS
    dsrc = pltpu.async_copy(ei_hbm.at[0, pl.ds(wid * EPT, EPT)], sall, gsem0)
    ddst = pltpu.async_copy(ei_hbm.at[1, pl.ds(wid * EPT, EPT)], dall, gsem1)
    dstg = pltpu.async_copy(h_hbm.at[pl.ds(r0, RPS), pl.ds(jb * H, H)],
                            hbuf.at[pl.ds(sid * RPS, RPS)], ssem0)

    zero16 = jnp.zeros((16,), jnp.float32)

    @pl.loop(0, ZR)
    def _zrow(i):
        zbuf[i, pl.ds(0, 16)] = zero16
        zbuf[i, pl.ds(16, 16)] = zero16

    zdesc = [pltpu.async_copy(zbuf, acc.at[pl.ds(sid * RPS + k * ZR, ZR)],
                              ssem1) for k in range(RPS // ZR)]
    dsrc.wait()
    ddst.wait()
    dstg.wait()
    for d in zdesc:
        d.wait()

    plsc.subcore_barrier()

    def fire_gather(w, b):
        pltpu.async_copy(hbuf.at[sall.at[pl.ds(w * EC, EC)]],
                         rows[b], gsem[b])

    def fire_scatter(w, b):
        pltpu.async_copy(rows[b], acc.at[dall.at[pl.ds(w * EC, EC)]],
                         ssem[b], add=True)

    fire_gather(0, 0)

    @pl.loop(0, NFC // 2)
    def _wave(j):
        for b in range(2):
            w = j * 2 + b
            nxt = jnp.where(w + 1 < NFC, w + 1, 0)

            @pl.when(w > 0)
            def _():
                pltpu.make_async_copy(
                    rows[1 - b], acc.at[dall.at[pl.ds(0, EC)]],
                    ssem[1 - b]).wait()

            fire_gather(nxt, 1 - b)
            pltpu.make_async_copy(
                hbuf.at[sall.at[pl.ds(0, EC)]], rows[b], gsem[b]).wait()
            fire_scatter(w, b)

    pltpu.make_async_copy(rows[1], acc.at[dall.at[pl.ds(0, EC)]],
                          ssem[1]).wait()
    pltpu.make_async_copy(hbuf.at[sall.at[pl.ds(0, EC)]], rows[0],
                          gsem[0]).wait()
    plsc.subcore_barrier()

    pltpu.sync_copy(acc.at[pl.ds(sid * RPS, RPS)], rows0.at[pl.ds(0, RPS)])
    pltpu.sync_copy(rows0.at[pl.ds(0, RPS)],
                    out_hbm.at[cid, pl.ds(r0, RPS), pl.ds(jb * H, H)])


@functools.cache
def _sc_aggregate_call():
    return pl.kernel(
        _sc_agg_body,
        out_type=jax.ShapeDtypeStruct((2, PR, 128), jnp.float32),
        mesh=plsc.VectorSubcoreMesh(core_axis_name="c", subcore_axis_name="s"),
        compiler_params=pltpu.CompilerParams(use_tc_tiling_on_sc=False),
        scratch_types=[
            pltpu.VMEM_SHARED((N_PAD, H), jnp.float32),
            pltpu.VMEM_SHARED((N_PAD, H), jnp.float32),
            pltpu.VMEM((EPT,), jnp.int32),
            pltpu.VMEM((EPT,), jnp.int32),
            pltpu.VMEM((EC, H), jnp.float32),
            pltpu.VMEM((EC, H), jnp.float32),
            pltpu.VMEM((ZR, H), jnp.float32),
            pltpu.SemaphoreType.DMA,
            pltpu.SemaphoreType.DMA,
            pltpu.SemaphoreType.DMA,
            pltpu.SemaphoreType.DMA,
        ],
    )


def _bn_relu(h, g, b):
    m = jnp.mean(h, axis=0, keepdims=True)
    v = jnp.mean((h - m) ** 2, axis=0, keepdims=True)
    return jnp.maximum((h - m) / jnp.sqrt(v + 1e-5) * g + b, 0.0)


def _mlp(h, w1, b1, g1, be1, w2, b2, g2, be2):
    h = _bn_relu(
        jnp.dot(h, w1[...], preferred_element_type=jnp.float32) + b1[...],
        g1[...], be1[...])
    h = _bn_relu(
        jnp.dot(h, w2[...], preferred_element_type=jnp.float32) + b2[...],
        g2[...], be2[...])
    return h


def _onehot(b_ref):
    ids = lax.broadcasted_iota(jnp.int32, (G, N), 0)
    return (b_ref[...] == ids).astype(jnp.float32)


def _pack_h(h):
    hp = jnp.concatenate([h, jnp.zeros((N_PAD - N, H), jnp.float32)], 0)
    return jnp.concatenate([hp[i * PR:(i + 1) * PR] for i in range(4)], 1)


def _unpack(hp):
    return jnp.concatenate([hp[:, i * H:(i + 1) * H] for i in range(4)], 0)


def _pool_out(h, b_ref, lw, lb, o_prev):
    oh = _onehot(b_ref)
    pooled = jnp.dot(oh, h, preferred_element_type=jnp.float32)
    cnt = jnp.dot(oh, jnp.ones((N, H), jnp.float32),
                  preferred_element_type=jnp.float32)
    pm = pooled / jnp.maximum(cnt, 1.0)
    return (o_prev +
            jnp.dot(pm, lw[...], preferred_element_type=jnp.float32)
            + lb[...])


def _first_body(x_ref, w1, b1, g1, be1, w2, b2, g2, be2, h_out):
    h = _mlp(x_ref[...], w1, b1, g1, be1, w2, b2, g2, be2)
    h_out[...] = _pack_h(h)


def _pool_body(h_ref, b_ref, lw, lb, o_prev, o_out):
    h = _unpack(h_ref[...])[:N]
    o_out[...] = _pool_out(h, b_ref, lw, lb, o_prev[...])


def _unpack_in(h_ref, agg_ref):
    s = h_ref[...] + agg_ref[0] + agg_ref[1]
    return _unpack(s)[:N]


def _conv_body(h_ref, agg_ref, w1, b1, g1, be1, w2, b2, g2, be2, h_out):
    h = _mlp(_unpack_in(h_ref, agg_ref), w1, b1, g1, be1, w2, b2, g2, be2)
    h_out[...] = _pack_h(h)


def _last_body(h_ref, agg_ref, b_ref, o_prev, w1, b1, g1, be1, w2, b2, g2,
               be2, lw, lb, o_out):
    h = _mlp(_unpack_in(h_ref, agg_ref), w1, b1, g1, be1, w2, b2, g2, be2)
    o_out[...] = _pool_out(h, b_ref, lw, lb, o_prev[...])


def _mlp_args(p):
    r = lambda a: a.reshape(1, -1)
    return (p["w1"], r(p["b1"]), r(p["g1"]), r(p["be1"]),
            p["w2"], r(p["b2"]), r(p["g2"]), r(p["be2"]))


_first_call = pl.pallas_call(
    _first_body,
    out_shape=jax.ShapeDtypeStruct((PR, 128), jnp.float32),
)

_pool_call = pl.pallas_call(
    _pool_body,
    out_shape=jax.ShapeDtypeStruct((G, T), jnp.float32),
)

_conv_call = pl.pallas_call(
    _conv_body,
    out_shape=jax.ShapeDtypeStruct((PR, 128), jnp.float32),
)

_last_call = pl.pallas_call(
    _last_body,
    out_shape=jax.ShapeDtypeStruct((G, T), jnp.float32),
)


@jax.jit
def kernel(x, edge_index, batch, params):
    b_row = batch.reshape(1, N)

    zero_gt = jnp.zeros((G, T), jnp.float32)
    h0 = _first_call(x, *_mlp_args(params["fh"]))
    sc_agg = _sc_aggregate_call()
    agg1 = sc_agg(edge_index, h0)
    out0 = _pool_call(h0, b_row, params["l0_w"],
                      params["l0_b"].reshape(1, T), zero_gt)
    h1 = _conv_call(h0, agg1, *_mlp_args(params["c1"]))
    agg2 = sc_agg(edge_index, h1)
    out01 = _pool_call(h1, b_row, params["l1_w"],
                       params["l1_b"].reshape(1, T), out0)
    return _last_call(h1, agg2, b_row, out01, *_mlp_args(params["c2"]),
                      params["l2_w"], params["l2_b"].reshape(1, T))
